# baseline (speedup 1.0000x reference)
import functools

import jax
import jax.numpy as jnp
from jax import lax
from jax.experimental import pallas as pl
from jax.experimental.pallas import tpu as pltpu
from jax.experimental.pallas import tpu_sc as plsc

NC = 2
NS = 16
L = 16


def _logits_body(x_ref, gw_ref, gb_ref, out_ref, xbf_ref):
    xb = x_ref[...]
    xbf_ref[...] = xb.astype(jnp.bfloat16)
    out_ref[...] = (
        jnp.dot(xb, gw_ref[...], preferred_element_type=jnp.float32)
        + gb_ref[...]
    )


def _routing_body(logits_ref, g_ref, lbuf, pbuf, actbuf, allbuf, shared,
                  *, rows_per_w, n_exp, tile_rows):
    c = lax.axis_index("c")
    s = lax.axis_index("s")
    wid = c * NS + s
    base = wid * rows_per_w
    w_per_block = tile_rows // rows_per_w

    pltpu.sync_copy(logits_ref.at[pl.ds(base * n_exp, rows_per_w * n_exp)], lbuf)

    iota = lax.iota(jnp.int32, L)
    zero = jnp.zeros((L,), jnp.float32)
    act = [zero] * n_exp
    for i in range(rows_per_w // L):
        rows8 = (i * L + iota) * n_exp
        ls = [plsc.load_gather(lbuf, [rows8 + e]) for e in range(n_exp)]
        mx = ls[0]
        for e in range(1, n_exp):
            mx = jnp.maximum(mx, ls[e])
        ps = [jnp.exp(l - mx) for l in ls]
        tot = ps[0]
        for e in range(1, n_exp):
            tot = tot + ps[e]
        ps = [p / tot for p in ps]
        pmax = ps[0]
        for e in range(1, n_exp):
            pmax = jnp.maximum(pmax, ps[e])
        nf = jnp.ones((L,), jnp.float32)
        for e in range(n_exp):
            oh = jnp.where(ps[e] == pmax, nf, 0.0)
            nf = nf - oh
            act[e] = jnp.maximum(act[e], oh)
        for e in range(n_exp):
            plsc.store_scatter(pbuf, [rows8 + e], ps[e])

    av = zero
    for e in range(n_exp):
        av = jnp.where(iota == e, lax.reduce_max(act[e], axes=(0,)), av)
    actbuf[...] = av
    pltpu.sync_copy(actbuf, shared.at[s])
    plsc.subcore_barrier()
    pltpu.sync_copy(shared, allbuf)

    sblk = (s // w_per_block) * w_per_block
    blk = allbuf[sblk, :]
    for j in range(1, w_per_block):
        blk = jnp.maximum(blk, allbuf[sblk + j, :])
    actbuf[...] = blk
    g16 = plsc.load_gather(actbuf, [jnp.remainder(iota, n_exp)])

    for j in range(rows_per_w * n_exp // L):
        pbuf[pl.ds(j * L, L)] = pbuf[pl.ds(j * L, L)] * g16
    pltpu.sync_copy(pbuf, g_ref.at[pl.ds(base * n_exp, rows_per_w * n_exp)])


def _moe_body(x_ref, w_ref, g_ref, out_ref, *, bn, n_exp):
    y = jnp.dot(x_ref[...], w_ref[...], preferred_element_type=jnp.float32)
    gb = g_ref[...]
    acc = None
    for e in range(n_exp):
        contrib = y[:, e * bn:(e + 1) * bn] * gb[:, e:e + 1]
        acc = contrib if acc is None else acc + contrib
    out_ref[...] = acc


def kernel(x, weight, gate_w, gate_b):
    B, d_model = x.shape
    n_exp = gate_w.shape[1]
    tile_rows = d_model
    assert B % tile_rows == 0
    n_row_blocks = B // tile_rows
    bn = min(512, d_model)
    n_tiles = d_model // bn
    rows_per_w = B // (NC * NS)

    gb2 = gate_b.reshape(1, n_exp)

    logits, xbf = pl.pallas_call(
        _logits_body,
        grid=(8,),
        in_specs=[
            pl.BlockSpec((B // 8, d_model), lambda m: (m, 0)),
            pl.BlockSpec((d_model, n_exp), lambda m: (0, 0)),
            pl.BlockSpec((1, n_exp), lambda m: (0, 0)),
        ],
        out_specs=[
            pl.BlockSpec((B // 8, n_exp), lambda m: (m, 0)),
            pl.BlockSpec((B // 8, d_model), lambda m: (m, 0)),
        ],
        out_shape=[
            jax.ShapeDtypeStruct((B, n_exp), jnp.float32),
            jax.ShapeDtypeStruct((B, d_model), jnp.bfloat16),
        ],
    )(x, gate_w, gb2)

    routing = functools.partial(
        _routing_body, rows_per_w=rows_per_w, n_exp=n_exp, tile_rows=tile_rows
    )
    g = pl.kernel(
        routing,
        out_type=jax.ShapeDtypeStruct((B * n_exp,), jnp.float32),
        mesh=plsc.VectorSubcoreMesh(
            core_axis_name="c", subcore_axis_name="s",
            num_cores=NC, num_subcores=NS,
        ),
        compiler_params=pltpu.CompilerParams(needs_layout_passes=False),
        scratch_types=[
            pltpu.VMEM((rows_per_w * n_exp,), jnp.float32),
            pltpu.VMEM((rows_per_w * n_exp,), jnp.float32),
            pltpu.VMEM((L,), jnp.float32),
            pltpu.VMEM((NS, L), jnp.float32),
            pltpu.VMEM_SHARED((NS, L), jnp.float32),
        ],
    )(logits.reshape(-1))
    g = g.reshape(B, n_exp)

    wp = (
        weight.reshape(d_model, n_exp, n_tiles, bn)
        .transpose(0, 2, 1, 3)
        .reshape(d_model, n_exp * d_model)
        .astype(jnp.bfloat16)
    )

    bm = 1024
    body = functools.partial(_moe_body, bn=bn, n_exp=n_exp)
    out = pl.pallas_call(
        body,
        grid=(B // bm, n_tiles),
        in_specs=[
            pl.BlockSpec((bm, d_model), lambda m, n: (m, 0)),
            pl.BlockSpec((d_model, n_exp * bn), lambda m, n: (0, n)),
            pl.BlockSpec((bm, n_exp), lambda m, n: (m, 0)),
        ],
        out_specs=pl.BlockSpec((bm, bn), lambda m, n: (m, n)),
        out_shape=jax.ShapeDtypeStruct((B, d_model), jnp.float32),
        compiler_params=pltpu.CompilerParams(
            dimension_semantics=("parallel", "parallel")
        ),
    )(xbf, wp, g)
    return out


# speedup vs baseline: 1.4442x; 1.4442x over previous
import functools

import jax
import jax.numpy as jnp
from jax import lax
from jax.experimental import pallas as pl
from jax.experimental.pallas import tpu as pltpu
from jax.experimental.pallas import tpu_sc as plsc

NC = 2
NS = 16
L = 16


def _logits_body(x_ref, gw_ref, gb_ref, out_ref):
    out_ref[...] = (
        jnp.dot(x_ref[...], gw_ref[...], preferred_element_type=jnp.float32)
        + gb_ref[...]
    )


def _routing_body(logits_ref, g_ref, lbuf, pbuf, actbuf, allbuf, shared,
                  *, rows_per_w, n_exp, tile_rows):
    c = lax.axis_index("c")
    s = lax.axis_index("s")
    wid = c * NS + s
    base = wid * rows_per_w
    w_per_block = tile_rows // rows_per_w

    pltpu.sync_copy(logits_ref.at[pl.ds(base * n_exp, rows_per_w * n_exp)], lbuf)

    iota = lax.iota(jnp.int32, L)
    zero = jnp.zeros((L,), jnp.float32)
    act = [zero] * n_exp
    for i in range(rows_per_w // L):
        rows8 = (i * L + iota) * n_exp
        ls = [plsc.load_gather(lbuf, [rows8 + e]) for e in range(n_exp)]
        mx = ls[0]
        for e in range(1, n_exp):
            mx = jnp.maximum(mx, ls[e])
        ps = [jnp.exp(l - mx) for l in ls]
        tot = ps[0]
        for e in range(1, n_exp):
            tot = tot + ps[e]
        ps = [p / tot for p in ps]
        pmax = ps[0]
        for e in range(1, n_exp):
            pmax = jnp.maximum(pmax, ps[e])
        nf = jnp.ones((L,), jnp.float32)
        for e in range(n_exp):
            oh = jnp.where(ps[e] == pmax, nf, 0.0)
            nf = nf - oh
            act[e] = jnp.maximum(act[e], oh)
        for e in range(n_exp):
            plsc.store_scatter(pbuf, [rows8 + e], ps[e])

    av = zero
    for e in range(n_exp):
        av = jnp.where(iota == e, lax.reduce_max(act[e], axes=(0,)), av)
    actbuf[...] = av
    pltpu.sync_copy(actbuf, shared.at[s])
    plsc.subcore_barrier()
    pltpu.sync_copy(shared, allbuf)

    sblk = (s // w_per_block) * w_per_block
    blk = allbuf[sblk, :]
    for j in range(1, w_per_block):
        blk = jnp.maximum(blk, allbuf[sblk + j, :])
    actbuf[...] = blk
    g16 = plsc.load_gather(actbuf, [jnp.remainder(iota, n_exp)])

    for j in range(rows_per_w * n_exp // L):
        pbuf[pl.ds(j * L, L)] = pbuf[pl.ds(j * L, L)] * g16
    pltpu.sync_copy(pbuf, g_ref.at[pl.ds(base * n_exp, rows_per_w * n_exp)])


def _moe_body(*refs, n_exp):
    x_ref = refs[0]
    w_refs = refs[1:1 + n_exp]
    g_ref = refs[1 + n_exp]
    out_ref = refs[2 + n_exp]
    xb = x_ref[...]
    gb = g_ref[...]
    acc = None
    for e in range(n_exp):
        y = jnp.dot(xb, w_refs[e][...], preferred_element_type=jnp.float32)
        contrib = y * gb[:, e:e + 1]
        acc = contrib if acc is None else acc + contrib
    out_ref[...] = acc


def kernel(x, weight, gate_w, gate_b):
    B, d_model = x.shape
    n_exp = gate_w.shape[1]
    tile_rows = d_model
    assert B % tile_rows == 0
    n_row_blocks = B // tile_rows
    bn = min(256, d_model)
    n_tiles = d_model // bn
    rows_per_w = B // (NC * NS)

    gb2 = gate_b.reshape(1, n_exp)

    logits = pl.pallas_call(
        _logits_body,
        grid=(8,),
        in_specs=[
            pl.BlockSpec((B // 8, d_model), lambda m: (m, 0)),
            pl.BlockSpec((d_model, n_exp), lambda m: (0, 0)),
            pl.BlockSpec((1, n_exp), lambda m: (0, 0)),
        ],
        out_specs=pl.BlockSpec((B // 8, n_exp), lambda m: (m, 0)),
        out_shape=jax.ShapeDtypeStruct((B, n_exp), jnp.float32),
    )(x, gate_w, gb2)

    routing = functools.partial(
        _routing_body, rows_per_w=rows_per_w, n_exp=n_exp, tile_rows=tile_rows
    )
    g = pl.kernel(
        routing,
        out_type=jax.ShapeDtypeStruct((B * n_exp,), jnp.float32),
        mesh=plsc.VectorSubcoreMesh(
            core_axis_name="c", subcore_axis_name="s",
            num_cores=NC, num_subcores=NS,
        ),
        compiler_params=pltpu.CompilerParams(needs_layout_passes=False),
        scratch_types=[
            pltpu.VMEM((rows_per_w * n_exp,), jnp.float32),
            pltpu.VMEM((rows_per_w * n_exp,), jnp.float32),
            pltpu.VMEM((L,), jnp.float32),
            pltpu.VMEM((NS, L), jnp.float32),
            pltpu.VMEM_SHARED((NS, L), jnp.float32),
        ],
    )(logits.reshape(-1))
    g = g.reshape(B, n_exp)

    bm = 1024
    body = functools.partial(_moe_body, n_exp=n_exp)
    w_specs = [
        pl.BlockSpec((d_model, bn), functools.partial(
            lambda e, m, n: (0, e * (d_model // bn) + n), e))
        for e in range(n_exp)
    ]
    out = pl.pallas_call(
        body,
        grid=(B // bm, n_tiles),
        in_specs=[pl.BlockSpec((bm, d_model), lambda m, n: (m, 0))]
        + w_specs
        + [pl.BlockSpec((bm, n_exp), lambda m, n: (m, 0))],
        out_specs=pl.BlockSpec((bm, bn), lambda m, n: (m, n)),
        out_shape=jax.ShapeDtypeStruct((B, d_model), jnp.float32),
        compiler_params=pltpu.CompilerParams(
            dimension_semantics=("parallel", "parallel")
        ),
    )(x, *([weight] * n_exp), g)
    return out
